# baseline (speedup 1.0000x reference)
import functools

import jax
import jax.numpy as jnp
from jax import lax
from jax.experimental import pallas as pl
from jax.experimental.pallas import tpu as pltpu
from jax.experimental.pallas import tpu_sc as plsc

_NW = 32
_D = 8
_CHUNK = 256

_GATHER_DNUMS = lax.GatherDimensionNumbers(
    offset_dims=(), collapsed_slice_dims=(0,), start_index_map=(0,)
)


def _vgather(vec, idx):
    return lax.gather(
        vec, idx[:, None], _GATHER_DNUMS, (1,),
        mode=lax.GatherScatterMode.PROMISE_IN_BOUNDS,
    )


@functools.lru_cache(maxsize=None)
def _gather_call(nb, ns):
    n_tok = nb * ns
    tpw = n_tok // _NW
    tiles_per_b = ns // tpw
    n_chunk = tpw // _CHUNK
    mesh = plsc.VectorSubcoreMesh(core_axis_name="c", subcore_axis_name="s")

    @functools.partial(
        pl.kernel,
        mesh=mesh,
        out_type=jax.ShapeDtypeStruct((_D, n_tok), jnp.float32),
        scratch_types=[
            pltpu.VMEM((_D, 16), jnp.float32),
            pltpu.VMEM((tpw,), jnp.int32),
            pltpu.VMEM((_D * _CHUNK,), jnp.float32),
            pltpu.VMEM((_D * _CHUNK,), jnp.float32),
            pltpu.SemaphoreType.DMA,
            pltpu.SemaphoreType.DMA,
        ],
    )
    def k(ids_hbm, t3_hbm, out_hbm, t3_v, ids_v, out_a, out_b, sem_a, sem_b):
        wid = lax.axis_index("s") * 2 + lax.axis_index("c")
        bidx = wid // tiles_per_b
        s0 = (wid % tiles_per_b) * tpw
        tok0 = bidx * ns + s0
        pltpu.sync_copy(t3_hbm, t3_v)
        pltpu.sync_copy(ids_hbm.at[bidx, pl.ds(s0, tpw)], ids_v)

        leaves = [t3_v[j, :] for j in range(_D)]

        bufs = (out_a, out_b)
        sems = (sem_a, sem_b)
        handles = [None, None]
        for cc in range(n_chunk):
            p = cc & 1
            if handles[p] is not None:
                for h in handles[p]:
                    h.wait()
            out_v = bufs[p]

            def group_body(g, carry, cc=cc, out_v=out_v):
                idv = ids_v[pl.ds(cc * _CHUNK + g * 16, 16)]
                for j in range(_D):
                    out_v[pl.ds(j * _CHUNK + g * 16, 16)] = _vgather(
                        leaves[j], idv
                    )
                return carry

            lax.fori_loop(0, _CHUNK // 16, group_body, 0)
            handles[p] = [
                pltpu.async_copy(
                    out_v.at[pl.ds(j * _CHUNK, _CHUNK)],
                    out_hbm.at[j, pl.ds(tok0 + cc * _CHUNK, _CHUNK)],
                    sems[p],
                )
                for j in range(_D)
            ]
        for hs in handles:
            if hs is not None:
                for h in hs:
                    h.wait()

    return k


def _table_body(w_proj_ref, w_emb_ref, b_ref, t3_ref):
    t_t = lax.dot_general(
        w_proj_ref[...], w_emb_ref[...],
        (((1,), (1,)), ((), ())),
        preferred_element_type=jnp.float32,
    ) + b_ref[...]
    t3_ref[...] = jnp.concatenate([t_t, t_t], axis=1)


def _fused_table3(W_emb, W_proj, b_proj):
    return pl.pallas_call(
        _table_body,
        out_shape=jax.ShapeDtypeStruct((_D, 16), jnp.float32),
    )(W_proj, W_emb, b_proj.reshape(_D, 1))


def _relayout_body(in_ref, out_ref):
    out_ref[...] = in_ref[...].T.reshape(out_ref.shape)


@functools.lru_cache(maxsize=None)
def _relayout_call(nb, ns):
    n_blk = _NW
    toks_per_blk = nb * ns // n_blk
    blks_per_b = n_blk // nb
    return pl.pallas_call(
        _relayout_body,
        grid=(n_blk,),
        in_specs=[pl.BlockSpec((_D, toks_per_blk), lambda i: (0, i))],
        out_specs=pl.BlockSpec(
            (1, toks_per_blk, _D),
            lambda i: (i // blks_per_b, i % blks_per_b, 0),
        ),
        out_shape=jax.ShapeDtypeStruct((nb, ns, _D), jnp.float32),
    )


def kernel(input_ids, W_emb, W_proj, b_proj):
    nb, ns = input_ids.shape
    ids = input_ids.astype(jnp.int32)
    t3 = _fused_table3(W_emb, W_proj, b_proj)
    stage = _gather_call(nb, ns)(ids, t3)
    return _relayout_call(nb, ns)(stage)


# speedup vs baseline: 2.3707x; 2.3707x over previous
import functools

import jax
import jax.numpy as jnp
from jax import lax
from jax.experimental import pallas as pl
from jax.experimental.pallas import tpu as pltpu
from jax.experimental.pallas import tpu_sc as plsc

_NW = 32
_D = 8
_CHUNK = 256

_GATHER_DNUMS = lax.GatherDimensionNumbers(
    offset_dims=(), collapsed_slice_dims=(0,), start_index_map=(0,)
)


def _vgather(vec, idx):
    return lax.gather(
        vec, idx[:, None], _GATHER_DNUMS, (1,),
        mode=lax.GatherScatterMode.PROMISE_IN_BOUNDS,
    )


@functools.lru_cache(maxsize=None)
def _gather_call(nb, ns):
    n_tok = nb * ns
    tpw = n_tok // _NW
    tiles_per_b = ns // tpw
    n_chunk = tpw // _CHUNK
    mesh = plsc.VectorSubcoreMesh(core_axis_name="c", subcore_axis_name="s")

    @functools.partial(
        pl.kernel,
        mesh=mesh,
        out_type=jax.ShapeDtypeStruct((nb * _D, ns), jnp.float32),
        scratch_types=[
            pltpu.VMEM((_D, 16), jnp.float32),
            pltpu.VMEM((tpw,), jnp.int32),
            pltpu.VMEM((_D * _CHUNK,), jnp.float32),
            pltpu.VMEM((_D * _CHUNK,), jnp.float32),
            pltpu.SemaphoreType.DMA,
            pltpu.SemaphoreType.DMA,
        ],
    )
    def k(ids_hbm, t3_hbm, out_hbm, t3_v, ids_v, out_a, out_b, sem_a, sem_b):
        wid = lax.axis_index("s") * 2 + lax.axis_index("c")
        bidx = wid // tiles_per_b
        s0 = (wid % tiles_per_b) * tpw
        pltpu.sync_copy(t3_hbm, t3_v)
        pltpu.sync_copy(ids_hbm.at[bidx, pl.ds(s0, tpw)], ids_v)

        leaves = [t3_v[j, :] for j in range(_D)]

        bufs = (out_a, out_b)
        sems = (sem_a, sem_b)
        handles = [None, None]
        for cc in range(n_chunk):
            p = cc & 1
            if handles[p] is not None:
                for h in handles[p]:
                    h.wait()
            out_v = bufs[p]

            def group_body(g, carry, cc=cc, out_v=out_v):
                idv = ids_v[pl.ds(cc * _CHUNK + g * 16, 16)]
                for j in range(_D):
                    out_v[pl.ds(j * _CHUNK + g * 16, 16)] = _vgather(
                        leaves[j], idv
                    )
                return carry

            lax.fori_loop(0, _CHUNK // 16, group_body, 0)
            handles[p] = [
                pltpu.async_copy(
                    out_v.at[pl.ds(j * _CHUNK, _CHUNK)],
                    out_hbm.at[bidx * _D + j,
                               pl.ds(s0 + cc * _CHUNK, _CHUNK)],
                    sems[p],
                )
                for j in range(_D)
            ]
        for hs in handles:
            if hs is not None:
                for h in hs:
                    h.wait()

    return k


def _table_body(w_proj_ref, w_emb_ref, b_ref, t3_ref):
    t_t = lax.dot_general(
        w_proj_ref[...], w_emb_ref[...],
        (((1,), (1,)), ((), ())),
        preferred_element_type=jnp.float32,
    ) + b_ref[...]
    t3_ref[...] = jnp.concatenate([t_t, t_t], axis=1)


def _fused_table3(W_emb, W_proj, b_proj):
    return pl.pallas_call(
        _table_body,
        out_shape=jax.ShapeDtypeStruct((_D, 16), jnp.float32),
    )(W_proj, W_emb, b_proj.reshape(_D, 1))


def kernel(input_ids, W_emb, W_proj, b_proj):
    nb, ns = input_ids.shape
    ids = input_ids.astype(jnp.int32)
    t3 = _fused_table3(W_emb, W_proj, b_proj)
    stage = _gather_call(nb, ns)(ids, t3)
    return stage.reshape(nb, _D, ns).transpose(0, 2, 1)


# speedup vs baseline: 2.3807x; 1.0042x over previous
import functools

import jax
import jax.numpy as jnp
from jax import lax
from jax.experimental import pallas as pl
from jax.experimental.pallas import tpu as pltpu
from jax.experimental.pallas import tpu_sc as plsc

_NW = 32
_D = 8
_CHUNK = 256

_GATHER_DNUMS = lax.GatherDimensionNumbers(
    offset_dims=(), collapsed_slice_dims=(0,), start_index_map=(0,)
)


def _vgather(vec, idx):
    return lax.gather(
        vec, idx[:, None], _GATHER_DNUMS, (1,),
        mode=lax.GatherScatterMode.PROMISE_IN_BOUNDS,
    )


@functools.lru_cache(maxsize=None)
def _gather_call(nb, ns):
    n_tok = nb * ns
    tpw = n_tok // _NW
    tiles_per_b = ns // tpw
    n_chunk = tpw // _CHUNK
    mesh = plsc.VectorSubcoreMesh(core_axis_name="c", subcore_axis_name="s")

    @functools.partial(
        pl.kernel,
        mesh=mesh,
        out_type=jax.ShapeDtypeStruct((nb * _D, ns), jnp.float32),
        scratch_types=[
            pltpu.VMEM((_D, 16), jnp.float32),
            pltpu.VMEM((tpw,), jnp.int32),
            pltpu.VMEM((_D * _CHUNK,), jnp.float32),
            pltpu.VMEM((_D * _CHUNK,), jnp.float32),
            pltpu.SemaphoreType.DMA,
            pltpu.SemaphoreType.DMA,
        ],
    )
    def k(ids_hbm, t3_hbm, out_hbm, t3_v, ids_v, out_a, out_b, sem_a, sem_b):
        wid = lax.axis_index("s") * 2 + lax.axis_index("c")
        bidx = wid // tiles_per_b
        s0 = (wid % tiles_per_b) * tpw
        pltpu.sync_copy(t3_hbm, t3_v)
        pltpu.sync_copy(ids_hbm.at[bidx, pl.ds(s0, tpw)], ids_v)

        leaves = [t3_v[j, :] for j in range(_D)]

        bufs = (out_a, out_b)
        sems = (sem_a, sem_b)
        handles = [None, None]
        for cc in range(n_chunk):
            p = cc & 1
            if handles[p] is not None:
                for h in handles[p]:
                    h.wait()
            out_v = bufs[p]

            def group_body(g, carry, cc=cc, out_v=out_v):
                idv = ids_v[pl.ds(cc * _CHUNK + g * 16, 16)]
                for j in range(_D):
                    out_v[pl.ds(j * _CHUNK + g * 16, 16)] = _vgather(
                        leaves[j], idv
                    )
                return carry

            lax.fori_loop(0, _CHUNK // 16, group_body, 0)
            handles[p] = [
                pltpu.async_copy(
                    out_v.at[pl.ds(j * _CHUNK, _CHUNK)],
                    out_hbm.at[bidx * _D + j,
                               pl.ds(s0 + cc * _CHUNK, _CHUNK)],
                    sems[p],
                )
                for j in range(_D)
            ]
        for hs in handles:
            if hs is not None:
                for h in hs:
                    h.wait()

    return k


def _table_body(w_emb_ref, w_proj_ref, b_ref, t3_ref):
    t = lax.dot_general(
        w_emb_ref[...], w_proj_ref[...],
        (((1,), (1,)), ((), ())),
        preferred_element_type=jnp.float32,
    ) + b_ref[...]
    t_t = t.T
    t3_ref[...] = jnp.concatenate([t_t, t_t], axis=1)


def _fused_table3(W_emb, W_proj, b_proj):
    return pl.pallas_call(
        _table_body,
        out_shape=jax.ShapeDtypeStruct((_D, 16), jnp.float32),
    )(W_emb, W_proj, b_proj.reshape(1, _D))


def kernel(input_ids, W_emb, W_proj, b_proj):
    nb, ns = input_ids.shape
    ids = input_ids.astype(jnp.int32)
    t3 = _fused_table3(W_emb, W_proj, b_proj)
    stage = _gather_call(nb, ns)(ids, t3)
    return stage.reshape(nb, _D, ns).transpose(0, 2, 1)


# speedup vs baseline: 2.4043x; 1.0099x over previous
import functools

import jax
import jax.numpy as jnp
from jax import lax
from jax.experimental import pallas as pl
from jax.experimental.pallas import tpu as pltpu
from jax.experimental.pallas import tpu_sc as plsc

_NW = 32
_D = 8
_CHUNK = 512

_GATHER_DNUMS = lax.GatherDimensionNumbers(
    offset_dims=(), collapsed_slice_dims=(0,), start_index_map=(0,)
)


def _vgather(vec, idx):
    return lax.gather(
        vec, idx[:, None], _GATHER_DNUMS, (1,),
        mode=lax.GatherScatterMode.PROMISE_IN_BOUNDS,
    )


@functools.lru_cache(maxsize=None)
def _gather_call(nb, ns):
    n_tok = nb * ns
    tpw = n_tok // _NW
    tiles_per_b = ns // tpw
    n_chunk = tpw // _CHUNK
    mesh = plsc.VectorSubcoreMesh(core_axis_name="c", subcore_axis_name="s")

    @functools.partial(
        pl.kernel,
        mesh=mesh,
        out_type=jax.ShapeDtypeStruct((nb * _D, ns), jnp.float32),
        scratch_types=[
            pltpu.VMEM((_D, 16), jnp.float32),
            pltpu.VMEM((tpw,), jnp.int32),
            pltpu.VMEM((_D * _CHUNK,), jnp.float32),
            pltpu.VMEM((_D * _CHUNK,), jnp.float32),
            pltpu.SemaphoreType.DMA,
            pltpu.SemaphoreType.DMA,
        ],
    )
    def k(ids_hbm, t3_hbm, out_hbm, t3_v, ids_v, out_a, out_b, sem_a, sem_b):
        wid = lax.axis_index("s") * 2 + lax.axis_index("c")
        bidx = wid // tiles_per_b
        s0 = (wid % tiles_per_b) * tpw
        pltpu.sync_copy(t3_hbm, t3_v)
        pltpu.sync_copy(ids_hbm.at[bidx, pl.ds(s0, tpw)], ids_v)

        leaves = [t3_v[j, :] for j in range(_D)]

        bufs = (out_a, out_b)
        sems = (sem_a, sem_b)
        handles = [None, None]
        for cc in range(n_chunk):
            p = cc & 1
            if handles[p] is not None:
                for h in handles[p]:
                    h.wait()
            out_v = bufs[p]

            def group_body(g2, carry, cc=cc, out_v=out_v):
                for u in range(2):
                    g = 2 * g2 + u
                    idv = ids_v[pl.ds(cc * _CHUNK + g * 16, 16)]
                    for j in range(_D):
                        out_v[pl.ds(j * _CHUNK + g * 16, 16)] = _vgather(
                            leaves[j], idv
                        )
                return carry

            lax.fori_loop(0, _CHUNK // 32, group_body, 0)
            handles[p] = [
                pltpu.async_copy(
                    out_v.at[pl.ds(j * _CHUNK, _CHUNK)],
                    out_hbm.at[bidx * _D + j,
                               pl.ds(s0 + cc * _CHUNK, _CHUNK)],
                    sems[p],
                )
                for j in range(_D)
            ]
        for hs in handles:
            if hs is not None:
                for h in hs:
                    h.wait()

    return k


def _table_body(w_emb_ref, w_proj_ref, b_ref, t3_ref):
    t = lax.dot_general(
        w_emb_ref[...], w_proj_ref[...],
        (((1,), (1,)), ((), ())),
        preferred_element_type=jnp.float32,
    ) + b_ref[...]
    t_t = t.T
    t3_ref[...] = jnp.concatenate([t_t, t_t], axis=1)


def _fused_table3(W_emb, W_proj, b_proj):
    return pl.pallas_call(
        _table_body,
        out_shape=jax.ShapeDtypeStruct((_D, 16), jnp.float32),
    )(W_emb, W_proj, b_proj.reshape(1, _D))


def kernel(input_ids, W_emb, W_proj, b_proj):
    nb, ns = input_ids.shape
    ids = input_ids.astype(jnp.int32)
    t3 = _fused_table3(W_emb, W_proj, b_proj)
    stage = _gather_call(nb, ns)(ids, t3)
    return stage.reshape(nb, _D, ns).transpose(0, 2, 1)
